# baseline (speedup 1.0000x reference)
import functools

import jax
import jax.numpy as jnp
from jax import lax
from jax.experimental import pallas as pl
from jax.experimental.pallas import tpu as pltpu
from jax.experimental.pallas import tpu_sc as plsc

N = 10000
E = 320000
T = 320000
HN = 128
HE = 16
EPS = 1e-5

NC = 2
NS = 16
NW = NC * NS

BK = 8000
NBLK = E // BK

_mesh = plsc.VectorSubcoreMesh(core_axis_name="c", subcore_axis_name="s")
_sc_params = pltpu.CompilerParams(use_tc_tiling_on_sc=False)


def _nodeproj_body(x_ref, w_ref, pi_ref, pj_ref, pk_ref):
    r = jnp.dot(x_ref[...], w_ref[...], preferred_element_type=jnp.float32)
    rb = r.astype(jnp.bfloat16)
    pi_ref[...] = rb[:, 0:32]
    pj_ref[...] = rb[:, 32:64]
    pk_ref[...] = rb[:, 64:96]


def _edgeproj_body(e_ref, wji_ref, wkj_ref, b_ref, qji_ref, qkj_ref):
    e = e_ref[...]
    qji_ref[...] = (jnp.dot(e, wji_ref[...], preferred_element_type=jnp.float32)
                    + b_ref[...]).astype(jnp.bfloat16)
    qkj_ref[...] = jnp.dot(e, wkj_ref[...],
                           preferred_element_type=jnp.float32).astype(jnp.bfloat16)


def _x2_body(prod_ref, w_ref, b_ref, x2_ref, st_ref, acc_ref):
    k = pl.program_id(0)
    x2 = jnp.dot(prod_ref[...], w_ref[...], preferred_element_type=jnp.float32) + b_ref[...]
    x2_ref[...] = x2

    @pl.when(k == 0)
    def _():
        acc_ref[...] = jnp.zeros_like(acc_ref)

    s = jnp.sum(x2, axis=0, keepdims=True)
    sq = jnp.sum(x2 * x2, axis=0, keepdims=True)
    acc_ref[...] = acc_ref[...] + jnp.concatenate([s, sq], axis=0)

    @pl.when(k == NBLK - 1)
    def _():
        st_ref[...] = acc_ref[...]


def _stats_body(x_ref, st_ref, acc_ref):
    k = pl.program_id(0)
    x = x_ref[...].astype(jnp.float32)

    @pl.when(k == 0)
    def _():
        acc_ref[...] = jnp.zeros_like(acc_ref)

    s = jnp.sum(x, axis=0, keepdims=True)
    sq = jnp.sum(x * x, axis=0, keepdims=True)
    acc_ref[...] = acc_ref[...] + jnp.concatenate([s, sq], axis=0)

    @pl.when(k == NBLK - 1)
    def _():
        st_ref[...] = acc_ref[...]


def _bn_act(x, st, g, b, nrows):
    mean = st[0:1, :] / nrows
    var = st[1:2, :] / nrows - mean * mean
    xn = g * (x - mean) * lax.rsqrt(var + EPS) + b
    f = xn[:, 0:HE]
    c = xn[:, HE:2 * HE]
    return jax.nn.sigmoid(f) * jnp.tanh(c)


def _act2_body(x2_ref, st_ref, g_ref, b_ref, y2_ref, sty_ref, acc_ref):
    k = pl.program_id(0)
    y = _bn_act(x2_ref[...], st_ref[...], g_ref[...], b_ref[...], float(E))
    y2_ref[...] = y

    @pl.when(k == 0)
    def _():
        acc_ref[...] = jnp.zeros_like(acc_ref)

    s = jnp.sum(y, axis=0, keepdims=True)
    sq = jnp.sum(y * y, axis=0, keepdims=True)
    acc_ref[...] = acc_ref[...] + jnp.concatenate([s, sq], axis=0)

    @pl.when(k == NBLK - 1)
    def _():
        sty_ref[...] = acc_ref[...]


def _msg_body(x3_ref, st_ref, g_ref, b_ref, msg_ref):
    y = _bn_act(x3_ref[...].astype(jnp.float32), st_ref[...], g_ref[...],
                b_ref[...], float(T))
    msg_ref[...] = y.astype(jnp.bfloat16)


def _final_body(e_ref, y2_ref, sty_ref, agg_ref, sta_ref, g22_ref, b22_ref,
                g32_ref, b32_ref, out_ref):
    sty = sty_ref[...]
    m2 = sty[0:1, :] / E
    v2 = sty[1:2, :] / E - m2 * m2
    c2e = g22_ref[...] * (y2_ref[...] - m2) * lax.rsqrt(v2 + EPS) + b22_ref[...]
    sta = sta_ref[...]
    m3 = sta[0:1, :] / E
    v3 = sta[1:2, :] / E - m3 * m3
    c3e = (g32_ref[...] * (agg_ref[...].astype(jnp.float32) - m3)
           * lax.rsqrt(v3 + EPS) + b32_ref[...])
    out_ref[...] = jnp.tanh(e_ref[...] + c2e + c3e)


KC = 200
RPW = T // NW
NCH_C = RPW // KC

KD = 40
NCH_D = RPW // KD


def _sc_gather5_body(pi, pj, pk, qji, qkj, ii, ij, ik, iji, ikj, out,
                     vi0, vi1, rr0, rr1, vout, si0, si1, sg0, sg1):
    w = lax.axis_index("s") * NC + lax.axis_index("c")
    tbls = (pi, pj, pk, qji, qkj)
    idxs = (ii, ij, ik, iji, ikj)

    def idx_descs(ch, vi, sem):
        base = w * RPW + ch * KC
        return [pltpu.make_async_copy(idxs[a].at[pl.ds(base, KC)],
                                      vi.at[a], sem)
                for a in range(5)]

    def gather_descs(vi, rr, sem):
        return [pltpu.make_async_copy(tbls[a].at[vi.at[a]], rr.at[a], sem)
                for a in range(5)]

    def fire(descs):
        for d in descs:
            d.start()

    def drain(descs):
        for d in descs:
            d.wait()

    def compute_store(ch, rr):
        def row(r, cr):
            sl = pl.ds(0, 32)
            vout[r, sl] = (rr[0, r, sl] + rr[1, r, sl] + rr[2, r, sl]
                           + rr[3, r, sl] + rr[4, r, sl])
            return cr
        lax.fori_loop(0, KC, row, 0)
        base = w * RPW + ch * KC
        pltpu.sync_copy(vout, out.at[pl.ds(base, KC)])

    fire(idx_descs(0, vi0, si0))
    drain(idx_descs(0, vi0, si0))
    fire(gather_descs(vi0, rr0, sg0))
    fire(idx_descs(1, vi1, si1))

    def pair(h, carry):
        ch_a = 2 * h
        drain(gather_descs(vi0, rr0, sg0))
        drain(idx_descs(ch_a + 1, vi1, si1))
        fire(gather_descs(vi1, rr1, sg1))
        compute_store(ch_a, rr0)
        nxt_a = jnp.minimum(ch_a + 2, NCH_C - 2)
        fire(idx_descs(nxt_a, vi0, si0))
        drain(idx_descs(nxt_a, vi0, si0))
        fire(gather_descs(vi0, rr0, sg0))
        drain(gather_descs(vi1, rr1, sg1))
        compute_store(ch_a + 1, rr1)
        nxt_b = jnp.minimum(ch_a + 3, NCH_C - 1)
        fire(idx_descs(nxt_b, vi1, si1))
        return carry

    lax.fori_loop(0, NCH_C // 2, pair, 0)
    drain(gather_descs(vi0, rr0, sg0))
    drain(idx_descs(NCH_C - 1, vi1, si1))


def _sc_pairprod_body(tbl, ii, ij, out, vi0, vi1, rr0, rr1, vout,
                      si0, si1, sg0, sg1):
    w = lax.axis_index("s") * NC + lax.axis_index("c")
    idxs = (ii, ij)

    def idx_descs(ch, vi, sem):
        base = w * RPW + ch * KD
        return [pltpu.make_async_copy(idxs[a].at[pl.ds(base, KD)],
                                      vi.at[a], sem)
                for a in range(2)]

    def gather_descs(vi, rr, sem):
        return [pltpu.make_async_copy(tbl.at[vi.at[a]], rr.at[a], sem)
                for a in range(2)]

    def fire(descs):
        for d in descs:
            d.start()

    def drain(descs):
        for d in descs:
            d.wait()

    def compute_store(ch, rr):
        def row(r, cr):
            for o in range(0, HN, 32):
                sl = pl.ds(o, 32)
                vout[r, sl] = rr[0, r, sl] * rr[1, r, sl]
            return cr
        lax.fori_loop(0, KD, row, 0)
        base = w * RPW + ch * KD
        pltpu.sync_copy(vout, out.at[pl.ds(base, KD)])

    fire(idx_descs(0, vi0, si0))
    drain(idx_descs(0, vi0, si0))
    fire(gather_descs(vi0, rr0, sg0))
    fire(idx_descs(1, vi1, si1))

    def pair(h, carry):
        ch_a = 2 * h
        drain(gather_descs(vi0, rr0, sg0))
        drain(idx_descs(ch_a + 1, vi1, si1))
        fire(gather_descs(vi1, rr1, sg1))
        compute_store(ch_a, rr0)
        nxt_a = jnp.minimum(ch_a + 2, NCH_D - 2)
        fire(idx_descs(nxt_a, vi0, si0))
        drain(idx_descs(nxt_a, vi0, si0))
        fire(gather_descs(vi0, rr0, sg0))
        drain(gather_descs(vi1, rr1, sg1))
        compute_store(ch_a + 1, rr1)
        nxt_b = jnp.minimum(ch_a + 3, NCH_D - 1)
        fire(idx_descs(nxt_b, vi1, si1))
        return carry

    lax.fori_loop(0, NCH_D // 2, pair, 0)
    drain(gather_descs(vi0, rr0, sg0))
    drain(idx_descs(NCH_D - 1, vi1, si1))


CH = 160000
DR = 128
SHR = CH + DR
ZR = SHR // NS
TPS = T // NS
BK3 = 2000
NSUB = BK3 // 80


def _sc_scatter_body(msg, iji, zeros, out, shr, vidx, vclamp, vmsg, sem):
    c = lax.axis_index("c")
    s = lax.axis_index("s")
    lo = c * CH
    pltpu.sync_copy(zeros, shr.at[pl.ds(s * ZR, ZR)])
    plsc.subcore_barrier()

    def big(b, carry):
        bb = s * TPS + b * BK3
        d0 = pltpu.async_copy(iji.at[pl.ds(bb, BK3)], vidx, sem)
        d1 = pltpu.async_copy(msg.at[pl.ds(bb, BK3)], vmsg, sem)
        d0.wait(); d1.wait()

        def sub(j, cr):
            for o in range(5):
                v = vidx[pl.ds(j * 80 + o * 16, 16)]
                inr = (v >= lo) & (v < lo + CH)
                dmp = CH + (v & (DR - 1))
                vclamp[j, pl.ds(o * 16, 16)] = jnp.where(inr, v - lo, dmp)
            return cr

        lax.fori_loop(0, NSUB, sub, 0)

        def scat_descs(j):
            return pltpu.make_async_copy(vmsg.at[pl.ds(j * 80, 80)],
                                         shr.at[vclamp.at[j]], sem)

        def scat_fire(j, cr):
            scat_descs(j).start(add=True)
            return cr

        def scat_drain(j, cr):
            scat_descs(j).wait()
            return cr

        lax.fori_loop(0, NSUB, scat_fire, 0)
        lax.fori_loop(0, NSUB, scat_drain, 0)
        return carry

    lax.fori_loop(0, TPS // BK3, big, 0)
    plsc.subcore_barrier()
    cpr = CH // NS
    pltpu.sync_copy(shr.at[pl.ds(s * cpr, cpr)],
                    out.at[pl.ds(lo + s * cpr, cpr)])


_sc_gather5 = functools.partial(
    pl.kernel,
    out_type=jax.ShapeDtypeStruct((T, 32), jnp.bfloat16),
    mesh=_mesh,
    compiler_params=_sc_params,
    scratch_types=(
        [pltpu.VMEM((5, KC), jnp.int32)] * 2
        + [pltpu.VMEM((5, KC, 32), jnp.bfloat16)] * 2
        + [pltpu.VMEM((KC, 32), jnp.bfloat16)]
        + [pltpu.SemaphoreType.DMA] * 4
    ),
)(_sc_gather5_body)

_sc_pairprod = functools.partial(
    pl.kernel,
    out_type=jax.ShapeDtypeStruct((E, HN), jnp.bfloat16),
    mesh=_mesh,
    compiler_params=_sc_params,
    scratch_types=(
        [pltpu.VMEM((2, KD), jnp.int32)] * 2
        + [pltpu.VMEM((2, KD, HN), jnp.bfloat16)] * 2
        + [pltpu.VMEM((KD, HN), jnp.bfloat16)]
        + [pltpu.SemaphoreType.DMA] * 4
    ),
)(_sc_pairprod_body)

_sc_scatter = functools.partial(
    pl.kernel,
    out_type=jax.ShapeDtypeStruct((E, HE), jnp.bfloat16),
    mesh=_mesh,
    compiler_params=_sc_params,
    scratch_types=(
        [pltpu.VMEM_SHARED((SHR, HE), jnp.bfloat16),
         pltpu.VMEM((BK3,), jnp.int32),
         pltpu.VMEM((NSUB, 80), jnp.int32),
         pltpu.VMEM((BK3, HE), jnp.bfloat16),
         pltpu.SemaphoreType.DMA]
    ),
)(_sc_scatter_body)


def kernel(node_emb, edge_emb, i, j, idx_i, idx_j, idx_k, idx_ji, idx_kj,
           W_c2, b_c2, W_c3, b_c3,
           g_c2, be_c2, g_c3, be_c3,
           g_c2_2, be_c2_2, g_c3_2, be_c3_2):
    f32 = jnp.float32
    wn = jnp.concatenate([W_c3[:, 0:HN].T, W_c3[:, HN:2 * HN].T,
                          W_c3[:, 2 * HN:3 * HN].T], axis=1)
    wji = W_c3[:, 3 * HN:3 * HN + HE].T
    wkj = W_c3[:, 3 * HN + HE:3 * HN + 2 * HE].T
    b3r = b_c3.reshape(1, 32)
    w2t = W_c2.T.astype(jnp.bfloat16)
    b2r = b_c2.reshape(1, 32)
    nb16 = node_emb.astype(jnp.bfloat16)

    p_i, p_j, p_k = pl.pallas_call(
        _nodeproj_body,
        out_shape=[jax.ShapeDtypeStruct((N, 32), jnp.bfloat16)] * 3,
    )(node_emb, wn)

    q_ji, q_kj = pl.pallas_call(
        _edgeproj_body,
        grid=(NBLK,),
        in_specs=[
            pl.BlockSpec((BK, HE), lambda k: (k, 0)),
            pl.BlockSpec((HE, 32), lambda k: (0, 0)),
            pl.BlockSpec((HE, 32), lambda k: (0, 0)),
            pl.BlockSpec((1, 32), lambda k: (0, 0)),
        ],
        out_specs=[pl.BlockSpec((BK, 32), lambda k: (k, 0))] * 2,
        out_shape=[jax.ShapeDtypeStruct((E, 32), jnp.bfloat16)] * 2,
    )(edge_emb, wji, wkj, b3r)

    x3 = _sc_gather5(p_i, p_j, p_k, q_ji, q_kj, idx_i, idx_j, idx_k,
                     idx_ji, idx_kj)

    prod = _sc_pairprod(nb16, i, j)

    x2, st_x2 = pl.pallas_call(
        _x2_body,
        grid=(NBLK,),
        in_specs=[
            pl.BlockSpec((BK, HN), lambda k: (k, 0)),
            pl.BlockSpec((HN, 32), lambda k: (0, 0)),
            pl.BlockSpec((1, 32), lambda k: (0, 0)),
        ],
        out_specs=[pl.BlockSpec((BK, 32), lambda k: (k, 0)),
                   pl.BlockSpec((2, 32), lambda k: (0, 0))],
        out_shape=[jax.ShapeDtypeStruct((E, 32), f32),
                   jax.ShapeDtypeStruct((2, 32), f32)],
        scratch_shapes=[pltpu.VMEM((2, 32), f32)],
    )(prod, w2t, b2r)

    st_x3 = pl.pallas_call(
        _stats_body,
        grid=(NBLK,),
        in_specs=[pl.BlockSpec((BK, 32), lambda k: (k, 0))],
        out_specs=pl.BlockSpec((2, 32), lambda k: (0, 0)),
        out_shape=jax.ShapeDtypeStruct((2, 32), f32),
        scratch_shapes=[pltpu.VMEM((2, 32), f32)],
    )(x3)

    y2, st_y2 = pl.pallas_call(
        _act2_body,
        grid=(NBLK,),
        in_specs=[
            pl.BlockSpec((BK, 32), lambda k: (k, 0)),
            pl.BlockSpec((2, 32), lambda k: (0, 0)),
            pl.BlockSpec((1, 32), lambda k: (0, 0)),
            pl.BlockSpec((1, 32), lambda k: (0, 0)),
        ],
        out_specs=[pl.BlockSpec((BK, HE), lambda k: (k, 0)),
                   pl.BlockSpec((2, HE), lambda k: (0, 0))],
        out_shape=[jax.ShapeDtypeStruct((E, HE), f32),
                   jax.ShapeDtypeStruct((2, HE), f32)],
        scratch_shapes=[pltpu.VMEM((2, HE), f32)],
    )(x2, st_x2, g_c2.reshape(1, 32), be_c2.reshape(1, 32))

    msg = pl.pallas_call(
        _msg_body,
        grid=(NBLK,),
        in_specs=[
            pl.BlockSpec((BK, 32), lambda k: (k, 0)),
            pl.BlockSpec((2, 32), lambda k: (0, 0)),
            pl.BlockSpec((1, 32), lambda k: (0, 0)),
            pl.BlockSpec((1, 32), lambda k: (0, 0)),
        ],
        out_specs=pl.BlockSpec((BK, HE), lambda k: (k, 0)),
        out_shape=jax.ShapeDtypeStruct((T, HE), jnp.bfloat16),
    )(x3, st_x3, g_c3.reshape(1, 32), be_c3.reshape(1, 32))

    zeros = jnp.zeros((ZR, HE), jnp.bfloat16)
    agg = _sc_scatter(msg, idx_ji, zeros)

    st_agg = pl.pallas_call(
        _stats_body,
        grid=(NBLK,),
        in_specs=[pl.BlockSpec((BK, HE), lambda k: (k, 0))],
        out_specs=pl.BlockSpec((2, HE), lambda k: (0, 0)),
        out_shape=jax.ShapeDtypeStruct((2, HE), f32),
        scratch_shapes=[pltpu.VMEM((2, HE), f32)],
    )(agg)

    out = pl.pallas_call(
        _final_body,
        grid=(NBLK,),
        in_specs=[
            pl.BlockSpec((BK, HE), lambda k: (k, 0)),
            pl.BlockSpec((BK, HE), lambda k: (k, 0)),
            pl.BlockSpec((2, HE), lambda k: (0, 0)),
            pl.BlockSpec((BK, HE), lambda k: (k, 0)),
            pl.BlockSpec((2, HE), lambda k: (0, 0)),
            pl.BlockSpec((1, HE), lambda k: (0, 0)),
            pl.BlockSpec((1, HE), lambda k: (0, 0)),
            pl.BlockSpec((1, HE), lambda k: (0, 0)),
            pl.BlockSpec((1, HE), lambda k: (0, 0)),
        ],
        out_specs=pl.BlockSpec((BK, HE), lambda k: (k, 0)),
        out_shape=jax.ShapeDtypeStruct((E, HE), f32),
    )(edge_emb, y2, st_y2, agg, st_agg,
      g_c2_2.reshape(1, HE), be_c2_2.reshape(1, HE),
      g_c3_2.reshape(1, HE), be_c3_2.reshape(1, HE))

    return out


# speedup vs baseline: 1.0527x; 1.0527x over previous
import functools

import numpy as np

import jax
import jax.numpy as jnp
from jax import lax
from jax.experimental import pallas as pl
from jax.experimental.pallas import tpu as pltpu
from jax.experimental.pallas import tpu_sc as plsc

N = 10000
E = 320000
T = 320000
HN = 128
HE = 16
EPS = 1e-5

NC = 2
NS = 16
NW = NC * NS

BK = 8000
NBLK = E // BK

_mesh = plsc.VectorSubcoreMesh(core_axis_name="c", subcore_axis_name="s")
_sc_params = pltpu.CompilerParams(use_tc_tiling_on_sc=False)
_sc_params_nolayout = pltpu.CompilerParams(use_tc_tiling_on_sc=False,
                                           needs_layout_passes=False)


def _nodeproj_body(x_ref, w_ref, pi_ref, pj_ref, pk_ref):
    r = jnp.dot(x_ref[...], w_ref[...], preferred_element_type=jnp.float32)
    rb = r.astype(jnp.bfloat16)
    pi_ref[...] = rb[:, 0:32]
    pj_ref[...] = rb[:, 32:64]
    pk_ref[...] = rb[:, 64:96]


def _edgeproj_body(e_ref, wji_ref, wkj_ref, b_ref, qji_ref, qkj_ref):
    e = e_ref[...]
    qji_ref[...] = (jnp.dot(e, wji_ref[...], preferred_element_type=jnp.float32)
                    + b_ref[...]).astype(jnp.bfloat16)
    qkj_ref[...] = jnp.dot(e, wkj_ref[...],
                           preferred_element_type=jnp.float32).astype(jnp.bfloat16)


def _x2_body(prod_ref, w_ref, b_ref, x2_ref, st_ref, acc_ref):
    k = pl.program_id(0)
    x2 = jnp.dot(prod_ref[...], w_ref[...], preferred_element_type=jnp.float32) + b_ref[...]
    x2_ref[...] = x2

    @pl.when(k == 0)
    def _():
        acc_ref[...] = jnp.zeros_like(acc_ref)

    s = jnp.sum(x2, axis=0, keepdims=True)
    sq = jnp.sum(x2 * x2, axis=0, keepdims=True)
    acc_ref[...] = acc_ref[...] + jnp.concatenate([s, sq], axis=0)

    @pl.when(k == NBLK - 1)
    def _():
        st_ref[...] = acc_ref[...]


def _stats_body(x_ref, st_ref, acc_ref):
    k = pl.program_id(0)
    x = x_ref[...].astype(jnp.float32)

    @pl.when(k == 0)
    def _():
        acc_ref[...] = jnp.zeros_like(acc_ref)

    s = jnp.sum(x, axis=0, keepdims=True)
    sq = jnp.sum(x * x, axis=0, keepdims=True)
    acc_ref[...] = acc_ref[...] + jnp.concatenate([s, sq], axis=0)

    @pl.when(k == NBLK - 1)
    def _():
        st_ref[...] = acc_ref[...]


def _bn_act(x, st, g, b, nrows):
    mean = st[0:1, :] / nrows
    var = st[1:2, :] / nrows - mean * mean
    xn = g * (x - mean) * lax.rsqrt(var + EPS) + b
    f = xn[:, 0:HE]
    c = xn[:, HE:2 * HE]
    return jax.nn.sigmoid(f) * jnp.tanh(c)


def _act2_body(x2_ref, st_ref, g_ref, b_ref, y2_ref, sty_ref, acc_ref):
    k = pl.program_id(0)
    y = _bn_act(x2_ref[...], st_ref[...], g_ref[...], b_ref[...], float(E))
    y2_ref[...] = y

    @pl.when(k == 0)
    def _():
        acc_ref[...] = jnp.zeros_like(acc_ref)

    s = jnp.sum(y, axis=0, keepdims=True)
    sq = jnp.sum(y * y, axis=0, keepdims=True)
    acc_ref[...] = acc_ref[...] + jnp.concatenate([s, sq], axis=0)

    @pl.when(k == NBLK - 1)
    def _():
        sty_ref[...] = acc_ref[...]


def _msg_body(x3_ref, st_ref, g_ref, b_ref, msg_ref):
    y = _bn_act(x3_ref[...].astype(jnp.float32), st_ref[...], g_ref[...],
                b_ref[...], float(T))
    msg_ref[...] = y.astype(jnp.bfloat16)


def _final_body(e_ref, y2_ref, sty_ref, agg_ref, sta_ref, g22_ref, b22_ref,
                g32_ref, b32_ref, out_ref):
    sty = sty_ref[...]
    m2 = sty[0:1, :] / E
    v2 = sty[1:2, :] / E - m2 * m2
    c2e = g22_ref[...] * (y2_ref[...] - m2) * lax.rsqrt(v2 + EPS) + b22_ref[...]
    sta = sta_ref[...]
    m3 = sta[0:1, :] / E
    v3 = sta[1:2, :] / E - m3 * m3
    c3e = (g32_ref[...] * (agg_ref[...].astype(jnp.float32) - m3)
           * lax.rsqrt(v3 + EPS) + b32_ref[...])
    out_ref[...] = jnp.tanh(e_ref[...] + c2e + c3e)


KC = 200
RPW = T // NW
NCH_C = RPW // KC

KD = 40
NCH_D = RPW // KD


def _sc_gather5_body(pi, pj, pk, qji, qkj, ii, ij, ik, iji, ikj, out,
                     vi0, vi1, rr0, rr1, vout, si0, si1, sg0, sg1):
    w = lax.axis_index("s") * NC + lax.axis_index("c")
    tbls = (pi, pj, pk, qji, qkj)
    idxs = (ii, ij, ik, iji, ikj)

    def idx_descs(ch, vi, sem):
        base = w * RPW + ch * KC
        return [pltpu.make_async_copy(idxs[a].at[pl.ds(base, KC)],
                                      vi.at[a], sem)
                for a in range(5)]

    def gather_descs(vi, rr, sem):
        return [pltpu.make_async_copy(tbls[a].at[vi.at[a]], rr.at[a], sem)
                for a in range(5)]

    def fire(descs):
        for d in descs:
            d.start()

    def drain(descs):
        for d in descs:
            d.wait()

    def compute_store(ch, rr):
        def row(r, cr):
            sl = pl.ds(0, 32)
            vout[r, sl] = (rr[0, r, sl] + rr[1, r, sl] + rr[2, r, sl]
                           + rr[3, r, sl] + rr[4, r, sl])
            return cr
        lax.fori_loop(0, KC, row, 0)
        base = w * RPW + ch * KC
        pltpu.sync_copy(vout, out.at[pl.ds(base, KC)])

    fire(idx_descs(0, vi0, si0))
    drain(idx_descs(0, vi0, si0))
    fire(gather_descs(vi0, rr0, sg0))
    fire(idx_descs(1, vi1, si1))

    def pair(h, carry):
        ch_a = 2 * h
        drain(gather_descs(vi0, rr0, sg0))
        drain(idx_descs(ch_a + 1, vi1, si1))
        fire(gather_descs(vi1, rr1, sg1))
        compute_store(ch_a, rr0)
        nxt_a = jnp.minimum(ch_a + 2, NCH_C - 2)
        fire(idx_descs(nxt_a, vi0, si0))
        drain(idx_descs(nxt_a, vi0, si0))
        fire(gather_descs(vi0, rr0, sg0))
        drain(gather_descs(vi1, rr1, sg1))
        compute_store(ch_a + 1, rr1)
        nxt_b = jnp.minimum(ch_a + 3, NCH_C - 1)
        fire(idx_descs(nxt_b, vi1, si1))
        return carry

    lax.fori_loop(0, NCH_C // 2, pair, 0)
    drain(gather_descs(vi0, rr0, sg0))
    drain(idx_descs(NCH_C - 1, vi1, si1))


def _sc_pairprod_body(tbl, ii, ij, out, vi0, vi1, rr0, rr1, vout,
                      si0, si1, sg0, sg1):
    w = lax.axis_index("s") * NC + lax.axis_index("c")
    idxs = (ii, ij)

    def idx_descs(ch, vi, sem):
        base = w * RPW + ch * KD
        return [pltpu.make_async_copy(idxs[a].at[pl.ds(base, KD)],
                                      vi.at[a], sem)
                for a in range(2)]

    def gather_descs(vi, rr, sem):
        return [pltpu.make_async_copy(tbl.at[vi.at[a]], rr.at[a], sem)
                for a in range(2)]

    def fire(descs):
        for d in descs:
            d.start()

    def drain(descs):
        for d in descs:
            d.wait()

    def compute_store(ch, rr):
        def row(r, cr):
            for o in range(0, HN, 32):
                sl = pl.ds(o, 32)
                a0, a1 = plsc.unpack(rr[0, r, sl],
                                     format=plsc.PackFormat.INTERLEAVED,
                                     preferred_element_type=jnp.float32)
                b0, b1 = plsc.unpack(rr[1, r, sl],
                                     format=plsc.PackFormat.INTERLEAVED,
                                     preferred_element_type=jnp.float32)
                vout[r, pl.ds(o, 16)] = a0 * b0
                vout[r, pl.ds(o + 16, 16)] = a1 * b1
            return cr
        lax.fori_loop(0, KD, row, 0)
        base = w * RPW + ch * KD
        pltpu.sync_copy(vout, out.at[pl.ds(base, KD)])

    fire(idx_descs(0, vi0, si0))
    drain(idx_descs(0, vi0, si0))
    fire(gather_descs(vi0, rr0, sg0))
    fire(idx_descs(1, vi1, si1))

    def pair(h, carry):
        ch_a = 2 * h
        drain(gather_descs(vi0, rr0, sg0))
        drain(idx_descs(ch_a + 1, vi1, si1))
        fire(gather_descs(vi1, rr1, sg1))
        compute_store(ch_a, rr0)
        nxt_a = jnp.minimum(ch_a + 2, NCH_D - 2)
        fire(idx_descs(nxt_a, vi0, si0))
        drain(idx_descs(nxt_a, vi0, si0))
        fire(gather_descs(vi0, rr0, sg0))
        drain(gather_descs(vi1, rr1, sg1))
        compute_store(ch_a + 1, rr1)
        nxt_b = jnp.minimum(ch_a + 3, NCH_D - 1)
        fire(idx_descs(nxt_b, vi1, si1))
        return carry

    lax.fori_loop(0, NCH_D // 2, pair, 0)
    drain(gather_descs(vi0, rr0, sg0))
    drain(idx_descs(NCH_D - 1, vi1, si1))


CH = 160000
DR = 128
SHR = CH + DR
ZR = SHR // NS
TPS = T // NS
BK3 = 2000
NSUB = BK3 // 80


def _sc_scatter_body(msg, iji, zeros, out, shr, vidx, vclamp, vmsg, sem):
    c = lax.axis_index("c")
    s = lax.axis_index("s")
    lo = c * CH
    pltpu.sync_copy(zeros, shr.at[pl.ds(s * ZR, ZR)])
    plsc.subcore_barrier()

    def big(b, carry):
        bb = s * TPS + b * BK3
        d0 = pltpu.async_copy(iji.at[pl.ds(bb, BK3)], vidx, sem)
        d1 = pltpu.async_copy(msg.at[pl.ds(bb, BK3)], vmsg, sem)
        d0.wait(); d1.wait()

        def sub(j, cr):
            for o in range(5):
                v = vidx[pl.ds(j * 80 + o * 16, 16)]
                inr = (v >= lo) & (v < lo + CH)
                dmp = CH + (v & (DR - 1))
                vclamp[j, pl.ds(o * 16, 16)] = jnp.where(inr, v - lo, dmp)
            return cr

        lax.fori_loop(0, NSUB, sub, 0)

        def scat_descs(j):
            return pltpu.make_async_copy(vmsg.at[pl.ds(j * 80, 80)],
                                         shr.at[vclamp.at[j]], sem)

        def scat_fire(j, cr):
            scat_descs(j).start(add=True)
            return cr

        def scat_drain(j, cr):
            scat_descs(j).wait()
            return cr

        lax.fori_loop(0, NSUB, scat_fire, 0)
        lax.fori_loop(0, NSUB, scat_drain, 0)
        return carry

    lax.fori_loop(0, TPS // BK3, big, 0)
    plsc.subcore_barrier()
    cpr = CH // NS
    pltpu.sync_copy(shr.at[pl.ds(s * cpr, cpr)],
                    out.at[pl.ds(lo + s * cpr, cpr)])


_sc_gather5 = functools.partial(
    pl.kernel,
    out_type=jax.ShapeDtypeStruct((T, 32), jnp.bfloat16),
    mesh=_mesh,
    compiler_params=_sc_params,
    scratch_types=(
        [pltpu.VMEM((5, KC), jnp.int32)] * 2
        + [pltpu.VMEM((5, KC, 32), jnp.bfloat16)] * 2
        + [pltpu.VMEM((KC, 32), jnp.bfloat16)]
        + [pltpu.SemaphoreType.DMA] * 4
    ),
)(_sc_gather5_body)

_sc_pairprod = functools.partial(
    pl.kernel,
    out_type=jax.ShapeDtypeStruct((E, HN), jnp.float32),
    mesh=_mesh,
    compiler_params=_sc_params_nolayout,
    scratch_types=(
        [pltpu.VMEM((2, KD), jnp.int32)] * 2
        + [pltpu.VMEM((2, KD, HN), jnp.bfloat16)] * 2
        + [pltpu.VMEM((KD, HN), jnp.float32)]
        + [pltpu.SemaphoreType.DMA] * 4
    ),
)(_sc_pairprod_body)

_sc_scatter = functools.partial(
    pl.kernel,
    out_type=jax.ShapeDtypeStruct((E, HE), jnp.bfloat16),
    mesh=_mesh,
    compiler_params=_sc_params,
    scratch_types=(
        [pltpu.VMEM_SHARED((SHR, HE), jnp.bfloat16),
         pltpu.VMEM((BK3,), jnp.int32),
         pltpu.VMEM((NSUB, 80), jnp.int32),
         pltpu.VMEM((BK3, HE), jnp.bfloat16),
         pltpu.SemaphoreType.DMA]
    ),
)(_sc_scatter_body)


def kernel(node_emb, edge_emb, i, j, idx_i, idx_j, idx_k, idx_ji, idx_kj,
           W_c2, b_c2, W_c3, b_c3,
           g_c2, be_c2, g_c3, be_c3,
           g_c2_2, be_c2_2, g_c3_2, be_c3_2):
    f32 = jnp.float32
    wn = jnp.concatenate([W_c3[:, 0:HN].T, W_c3[:, HN:2 * HN].T,
                          W_c3[:, 2 * HN:3 * HN].T], axis=1)
    wji = W_c3[:, 3 * HN:3 * HN + HE].T
    wkj = W_c3[:, 3 * HN + HE:3 * HN + 2 * HE].T
    b3r = b_c3.reshape(1, 32)
    perm = np.concatenate(
        [np.concatenate([np.arange(g * 32, g * 32 + 32, 2),
                         np.arange(g * 32 + 1, g * 32 + 32, 2)])
         for g in range(HN // 32)])
    w2t = W_c2.T[perm, :]
    b2r = b_c2.reshape(1, 32)
    nb16 = node_emb.astype(jnp.bfloat16)

    p_i, p_j, p_k = pl.pallas_call(
        _nodeproj_body,
        out_shape=[jax.ShapeDtypeStruct((N, 32), jnp.bfloat16)] * 3,
    )(node_emb, wn)

    q_ji, q_kj = pl.pallas_call(
        _edgeproj_body,
        grid=(NBLK,),
        in_specs=[
            pl.BlockSpec((BK, HE), lambda k: (k, 0)),
            pl.BlockSpec((HE, 32), lambda k: (0, 0)),
            pl.BlockSpec((HE, 32), lambda k: (0, 0)),
            pl.BlockSpec((1, 32), lambda k: (0, 0)),
        ],
        out_specs=[pl.BlockSpec((BK, 32), lambda k: (k, 0))] * 2,
        out_shape=[jax.ShapeDtypeStruct((E, 32), jnp.bfloat16)] * 2,
    )(edge_emb, wji, wkj, b3r)

    x3 = _sc_gather5(p_i, p_j, p_k, q_ji, q_kj, idx_i, idx_j, idx_k,
                     idx_ji, idx_kj)

    prod = _sc_pairprod(nb16, i, j)

    x2, st_x2 = pl.pallas_call(
        _x2_body,
        grid=(NBLK,),
        in_specs=[
            pl.BlockSpec((BK, HN), lambda k: (k, 0)),
            pl.BlockSpec((HN, 32), lambda k: (0, 0)),
            pl.BlockSpec((1, 32), lambda k: (0, 0)),
        ],
        out_specs=[pl.BlockSpec((BK, 32), lambda k: (k, 0)),
                   pl.BlockSpec((2, 32), lambda k: (0, 0))],
        out_shape=[jax.ShapeDtypeStruct((E, 32), f32),
                   jax.ShapeDtypeStruct((2, 32), f32)],
        scratch_shapes=[pltpu.VMEM((2, 32), f32)],
    )(prod, w2t, b2r)

    st_x3 = pl.pallas_call(
        _stats_body,
        grid=(NBLK,),
        in_specs=[pl.BlockSpec((BK, 32), lambda k: (k, 0))],
        out_specs=pl.BlockSpec((2, 32), lambda k: (0, 0)),
        out_shape=jax.ShapeDtypeStruct((2, 32), f32),
        scratch_shapes=[pltpu.VMEM((2, 32), f32)],
    )(x3)

    y2, st_y2 = pl.pallas_call(
        _act2_body,
        grid=(NBLK,),
        in_specs=[
            pl.BlockSpec((BK, 32), lambda k: (k, 0)),
            pl.BlockSpec((2, 32), lambda k: (0, 0)),
            pl.BlockSpec((1, 32), lambda k: (0, 0)),
            pl.BlockSpec((1, 32), lambda k: (0, 0)),
        ],
        out_specs=[pl.BlockSpec((BK, HE), lambda k: (k, 0)),
                   pl.BlockSpec((2, HE), lambda k: (0, 0))],
        out_shape=[jax.ShapeDtypeStruct((E, HE), f32),
                   jax.ShapeDtypeStruct((2, HE), f32)],
        scratch_shapes=[pltpu.VMEM((2, HE), f32)],
    )(x2, st_x2, g_c2.reshape(1, 32), be_c2.reshape(1, 32))

    msg = pl.pallas_call(
        _msg_body,
        grid=(NBLK,),
        in_specs=[
            pl.BlockSpec((BK, 32), lambda k: (k, 0)),
            pl.BlockSpec((2, 32), lambda k: (0, 0)),
            pl.BlockSpec((1, 32), lambda k: (0, 0)),
            pl.BlockSpec((1, 32), lambda k: (0, 0)),
        ],
        out_specs=pl.BlockSpec((BK, HE), lambda k: (k, 0)),
        out_shape=jax.ShapeDtypeStruct((T, HE), jnp.bfloat16),
    )(x3, st_x3, g_c3.reshape(1, 32), be_c3.reshape(1, 32))

    zeros = jnp.zeros((ZR, HE), jnp.bfloat16)
    agg = _sc_scatter(msg, idx_ji, zeros)

    st_agg = pl.pallas_call(
        _stats_body,
        grid=(NBLK,),
        in_specs=[pl.BlockSpec((BK, HE), lambda k: (k, 0))],
        out_specs=pl.BlockSpec((2, HE), lambda k: (0, 0)),
        out_shape=jax.ShapeDtypeStruct((2, HE), f32),
        scratch_shapes=[pltpu.VMEM((2, HE), f32)],
    )(agg)

    out = pl.pallas_call(
        _final_body,
        grid=(NBLK,),
        in_specs=[
            pl.BlockSpec((BK, HE), lambda k: (k, 0)),
            pl.BlockSpec((BK, HE), lambda k: (k, 0)),
            pl.BlockSpec((2, HE), lambda k: (0, 0)),
            pl.BlockSpec((BK, HE), lambda k: (k, 0)),
            pl.BlockSpec((2, HE), lambda k: (0, 0)),
            pl.BlockSpec((1, HE), lambda k: (0, 0)),
            pl.BlockSpec((1, HE), lambda k: (0, 0)),
            pl.BlockSpec((1, HE), lambda k: (0, 0)),
            pl.BlockSpec((1, HE), lambda k: (0, 0)),
        ],
        out_specs=pl.BlockSpec((BK, HE), lambda k: (k, 0)),
        out_shape=jax.ShapeDtypeStruct((E, HE), f32),
    )(edge_emb, y2, st_y2, agg, st_agg,
      g_c2_2.reshape(1, HE), be_c2_2.reshape(1, HE),
      g_c3_2.reshape(1, HE), be_c3_2.reshape(1, HE))

    return out


# speedup vs baseline: 1.1125x; 1.0569x over previous
import functools

import jax
import jax.numpy as jnp
from jax import lax
from jax.experimental import pallas as pl
from jax.experimental.pallas import tpu as pltpu
from jax.experimental.pallas import tpu_sc as plsc

N = 10000
E = 320000
T = 320000
HN = 128
HE = 16
EPS = 1e-5

NC = 2
NS = 16
NW = NC * NS

BK = 8000
NBLK = E // BK

_mesh = plsc.VectorSubcoreMesh(core_axis_name="c", subcore_axis_name="s")
_sc_params = pltpu.CompilerParams(use_tc_tiling_on_sc=False)
_sc_params_nolayout = pltpu.CompilerParams(use_tc_tiling_on_sc=False,
                                           needs_layout_passes=False)


def _nodeproj_body(x_ref, w_ref, pi_ref, pj_ref, pk_ref):
    r = jnp.dot(x_ref[...], w_ref[...], preferred_element_type=jnp.float32)
    pi_ref[...] = r[:, 0:32]
    pj_ref[...] = r[:, 32:64]
    pk_ref[...] = r[:, 64:96]


def _edgeproj_body(e_ref, wji_ref, wkj_ref, b_ref, qji_ref, qkj_ref):
    e = e_ref[...]
    qji_ref[...] = jnp.dot(e, wji_ref[...], preferred_element_type=jnp.float32) + b_ref[...]
    qkj_ref[...] = jnp.dot(e, wkj_ref[...], preferred_element_type=jnp.float32)


def _x2_body(prod_ref, w_ref, b_ref, x2_ref, st_ref, acc_ref):
    k = pl.program_id(0)
    x2 = jnp.dot(prod_ref[...], w_ref[...], preferred_element_type=jnp.float32) + b_ref[...]
    x2_ref[...] = x2

    @pl.when(k == 0)
    def _():
        acc_ref[...] = jnp.zeros_like(acc_ref)

    s = jnp.sum(x2, axis=0, keepdims=True)
    sq = jnp.sum(x2 * x2, axis=0, keepdims=True)
    acc_ref[...] = acc_ref[...] + jnp.concatenate([s, sq], axis=0)

    @pl.when(k == NBLK - 1)
    def _():
        st_ref[...] = acc_ref[...]


def _stats_body(x_ref, st_ref, acc_ref):
    k = pl.program_id(0)
    x = x_ref[...].astype(jnp.float32)

    @pl.when(k == 0)
    def _():
        acc_ref[...] = jnp.zeros_like(acc_ref)

    s = jnp.sum(x, axis=0, keepdims=True)
    sq = jnp.sum(x * x, axis=0, keepdims=True)
    acc_ref[...] = acc_ref[...] + jnp.concatenate([s, sq], axis=0)

    @pl.when(k == NBLK - 1)
    def _():
        st_ref[...] = acc_ref[...]


def _bn_act(x, st, g, b, nrows):
    mean = st[0:1, :] / nrows
    var = st[1:2, :] / nrows - mean * mean
    xn = g * (x - mean) * lax.rsqrt(var + EPS) + b
    f = xn[:, 0:HE]
    c = xn[:, HE:2 * HE]
    return jax.nn.sigmoid(f) * jnp.tanh(c)


def _act2_body(x2_ref, st_ref, g_ref, b_ref, y2_ref, sty_ref, acc_ref):
    k = pl.program_id(0)
    y = _bn_act(x2_ref[...], st_ref[...], g_ref[...], b_ref[...], float(E))
    y2_ref[...] = y

    @pl.when(k == 0)
    def _():
        acc_ref[...] = jnp.zeros_like(acc_ref)

    s = jnp.sum(y, axis=0, keepdims=True)
    sq = jnp.sum(y * y, axis=0, keepdims=True)
    acc_ref[...] = acc_ref[...] + jnp.concatenate([s, sq], axis=0)

    @pl.when(k == NBLK - 1)
    def _():
        sty_ref[...] = acc_ref[...]


def _msg_body(x3_ref, st_ref, g_ref, b_ref, msg_ref):
    y = _bn_act(x3_ref[...].astype(jnp.float32), st_ref[...], g_ref[...],
                b_ref[...], float(T))
    msg_ref[...] = y.astype(jnp.bfloat16)


def _final_body(e_ref, y2_ref, sty_ref, agg_ref, sta_ref, g22_ref, b22_ref,
                g32_ref, b32_ref, out_ref):
    sty = sty_ref[...]
    m2 = sty[0:1, :] / E
    v2 = sty[1:2, :] / E - m2 * m2
    c2e = g22_ref[...] * (y2_ref[...] - m2) * lax.rsqrt(v2 + EPS) + b22_ref[...]
    sta = sta_ref[...]
    m3 = sta[0:1, :] / E
    v3 = sta[1:2, :] / E - m3 * m3
    c3e = (g32_ref[...] * (agg_ref[...].astype(jnp.float32) - m3)
           * lax.rsqrt(v3 + EPS) + b32_ref[...])
    out_ref[...] = jnp.tanh(e_ref[...] + c2e + c3e)


KC = 200
RPW = T // NW
NCH_C = RPW // KC

KD = 40
NCH_D = RPW // KD


def _sc_gather5_body(pi, pj, pk, qji, qkj, ii, ij, ik, iji, ikj, out,
                     vi0, vi1, rr0, rr1, vout, si0, si1, sg0, sg1):
    w = lax.axis_index("s") * NC + lax.axis_index("c")
    tbls = (pi, pj, pk, qji, qkj)
    idxs = (ii, ij, ik, iji, ikj)

    def idx_descs(ch, vi, sem):
        base = w * RPW + ch * KC
        return [pltpu.make_async_copy(idxs[a].at[pl.ds(base, KC)],
                                      vi.at[a], sem)
                for a in range(5)]

    def gather_descs(vi, rr, sem):
        return [pltpu.make_async_copy(tbls[a].at[vi.at[a]], rr.at[a], sem)
                for a in range(5)]

    def fire(descs):
        for d in descs:
            d.start()

    def drain(descs):
        for d in descs:
            d.wait()

    def compute_store(ch, rr):
        def row(r, cr):
            for o in (0, 16):
                sl = pl.ds(o, 16)
                vout[r, sl] = (rr[0, r, sl] + rr[1, r, sl] + rr[2, r, sl]
                               + rr[3, r, sl] + rr[4, r, sl])
            return cr
        lax.fori_loop(0, KC, row, 0)
        base = w * RPW + ch * KC
        pltpu.sync_copy(vout, out.at[pl.ds(base, KC)])

    fire(idx_descs(0, vi0, si0))
    drain(idx_descs(0, vi0, si0))
    fire(gather_descs(vi0, rr0, sg0))
    fire(idx_descs(1, vi1, si1))

    def pair(h, carry):
        ch_a = 2 * h
        drain(gather_descs(vi0, rr0, sg0))
        drain(idx_descs(ch_a + 1, vi1, si1))
        fire(gather_descs(vi1, rr1, sg1))
        compute_store(ch_a, rr0)
        nxt_a = jnp.minimum(ch_a + 2, NCH_C - 2)
        fire(idx_descs(nxt_a, vi0, si0))
        drain(idx_descs(nxt_a, vi0, si0))
        fire(gather_descs(vi0, rr0, sg0))
        drain(gather_descs(vi1, rr1, sg1))
        compute_store(ch_a + 1, rr1)
        nxt_b = jnp.minimum(ch_a + 3, NCH_C - 1)
        fire(idx_descs(nxt_b, vi1, si1))
        return carry

    lax.fori_loop(0, NCH_C // 2, pair, 0)
    drain(gather_descs(vi0, rr0, sg0))
    drain(idx_descs(NCH_C - 1, vi1, si1))


def _sc_pairprod_body(tbl, ii, ij, out, vi0, vi1, rr0, rr1, vout,
                      si0, si1, sg0, sg1):
    w = lax.axis_index("s") * NC + lax.axis_index("c")
    idxs = (ii, ij)

    def idx_descs(ch, vi, sem):
        base = w * RPW + ch * KD
        return [pltpu.make_async_copy(idxs[a].at[pl.ds(base, KD)],
                                      vi.at[a], sem)
                for a in range(2)]

    def gather_descs(vi, rr, sem):
        return [pltpu.make_async_copy(tbl.at[vi.at[a]], rr.at[a], sem)
                for a in range(2)]

    def fire(descs):
        for d in descs:
            d.start()

    def drain(descs):
        for d in descs:
            d.wait()

    def compute_store(ch, rr):
        def row(r, cr):
            for o in range(0, HN, 16):
                sl = pl.ds(o, 16)
                vout[r, sl] = rr[0, r, sl] * rr[1, r, sl]
            return cr
        lax.fori_loop(0, KD, row, 0)
        base = w * RPW + ch * KD
        pltpu.sync_copy(vout, out.at[pl.ds(base, KD)])

    fire(idx_descs(0, vi0, si0))
    drain(idx_descs(0, vi0, si0))
    fire(gather_descs(vi0, rr0, sg0))
    fire(idx_descs(1, vi1, si1))

    def pair(h, carry):
        ch_a = 2 * h
        drain(gather_descs(vi0, rr0, sg0))
        drain(idx_descs(ch_a + 1, vi1, si1))
        fire(gather_descs(vi1, rr1, sg1))
        compute_store(ch_a, rr0)
        nxt_a = jnp.minimum(ch_a + 2, NCH_D - 2)
        fire(idx_descs(nxt_a, vi0, si0))
        drain(idx_descs(nxt_a, vi0, si0))
        fire(gather_descs(vi0, rr0, sg0))
        drain(gather_descs(vi1, rr1, sg1))
        compute_store(ch_a + 1, rr1)
        nxt_b = jnp.minimum(ch_a + 3, NCH_D - 1)
        fire(idx_descs(nxt_b, vi1, si1))
        return carry

    lax.fori_loop(0, NCH_D // 2, pair, 0)
    drain(gather_descs(vi0, rr0, sg0))
    drain(idx_descs(NCH_D - 1, vi1, si1))


CH = 160000
DR = 128
SHR = CH + DR
ZR = SHR // NS
TPS = T // NS
BK3 = 2000
NSUB = BK3 // 80


def _sc_scatter_body(msg, iji, zeros, out, shr, vidx, vclamp, vmsg, sem):
    c = lax.axis_index("c")
    s = lax.axis_index("s")
    lo = c * CH
    pltpu.sync_copy(zeros, shr.at[pl.ds(s * ZR, ZR)])
    plsc.subcore_barrier()

    def big(b, carry):
        bb = s * TPS + b * BK3
        d0 = pltpu.async_copy(iji.at[pl.ds(bb, BK3)], vidx, sem)
        d1 = pltpu.async_copy(msg.at[pl.ds(bb, BK3)], vmsg, sem)
        d0.wait(); d1.wait()

        def sub(j, cr):
            for o in range(5):
                v = vidx[pl.ds(j * 80 + o * 16, 16)]
                inr = (v >= lo) & (v < lo + CH)
                dmp = CH + (v & (DR - 1))
                vclamp[j, pl.ds(o * 16, 16)] = jnp.where(inr, v - lo, dmp)
            return cr

        lax.fori_loop(0, NSUB, sub, 0)

        def scat_descs(j):
            return pltpu.make_async_copy(vmsg.at[pl.ds(j * 80, 80)],
                                         shr.at[vclamp.at[j]], sem)

        def scat_fire(j, cr):
            scat_descs(j).start(add=True)
            return cr

        def scat_drain(j, cr):
            scat_descs(j).wait()
            return cr

        lax.fori_loop(0, NSUB, scat_fire, 0)
        lax.fori_loop(0, NSUB, scat_drain, 0)
        return carry

    lax.fori_loop(0, TPS // BK3, big, 0)
    plsc.subcore_barrier()
    cpr = CH // NS
    pltpu.sync_copy(shr.at[pl.ds(s * cpr, cpr)],
                    out.at[pl.ds(lo + s * cpr, cpr)])


_sc_gather5 = functools.partial(
    pl.kernel,
    out_type=jax.ShapeDtypeStruct((T, 32), jnp.float32),
    mesh=_mesh,
    compiler_params=_sc_params,
    scratch_types=(
        [pltpu.VMEM((5, KC), jnp.int32)] * 2
        + [pltpu.VMEM((5, KC, 32), jnp.float32)] * 2
        + [pltpu.VMEM((KC, 32), jnp.float32)]
        + [pltpu.SemaphoreType.DMA] * 4
    ),
)(_sc_gather5_body)

_sc_pairprod = functools.partial(
    pl.kernel,
    out_type=jax.ShapeDtypeStruct((E, HN), jnp.float32),
    mesh=_mesh,
    compiler_params=_sc_params,
    scratch_types=(
        [pltpu.VMEM((2, KD), jnp.int32)] * 2
        + [pltpu.VMEM((2, KD, HN), jnp.float32)] * 2
        + [pltpu.VMEM((KD, HN), jnp.float32)]
        + [pltpu.SemaphoreType.DMA] * 4
    ),
)(_sc_pairprod_body)

_sc_scatter = functools.partial(
    pl.kernel,
    out_type=jax.ShapeDtypeStruct((E, HE), jnp.bfloat16),
    mesh=_mesh,
    compiler_params=_sc_params,
    scratch_types=(
        [pltpu.VMEM_SHARED((SHR, HE), jnp.bfloat16),
         pltpu.VMEM((BK3,), jnp.int32),
         pltpu.VMEM((NSUB, 80), jnp.int32),
         pltpu.VMEM((BK3, HE), jnp.bfloat16),
         pltpu.SemaphoreType.DMA]
    ),
)(_sc_scatter_body)


def kernel(node_emb, edge_emb, i, j, idx_i, idx_j, idx_k, idx_ji, idx_kj,
           W_c2, b_c2, W_c3, b_c3,
           g_c2, be_c2, g_c3, be_c3,
           g_c2_2, be_c2_2, g_c3_2, be_c3_2):
    f32 = jnp.float32
    wn = jnp.concatenate([W_c3[:, 0:HN].T, W_c3[:, HN:2 * HN].T,
                          W_c3[:, 2 * HN:3 * HN].T], axis=1)
    wji = W_c3[:, 3 * HN:3 * HN + HE].T
    wkj = W_c3[:, 3 * HN + HE:3 * HN + 2 * HE].T
    b3r = b_c3.reshape(1, 32)
    w2t = W_c2.T
    b2r = b_c2.reshape(1, 32)

    p_i, p_j, p_k = pl.pallas_call(
        _nodeproj_body,
        out_shape=[jax.ShapeDtypeStruct((N, 32), f32)] * 3,
    )(node_emb, wn)

    q_ji, q_kj = pl.pallas_call(
        _edgeproj_body,
        grid=(NBLK,),
        in_specs=[
            pl.BlockSpec((BK, HE), lambda k: (k, 0)),
            pl.BlockSpec((HE, 32), lambda k: (0, 0)),
            pl.BlockSpec((HE, 32), lambda k: (0, 0)),
            pl.BlockSpec((1, 32), lambda k: (0, 0)),
        ],
        out_specs=[pl.BlockSpec((BK, 32), lambda k: (k, 0))] * 2,
        out_shape=[jax.ShapeDtypeStruct((E, 32), f32)] * 2,
    )(edge_emb, wji, wkj, b3r)

    x3 = _sc_gather5(p_i, p_j, p_k, q_ji, q_kj, idx_i, idx_j, idx_k,
                     idx_ji, idx_kj)

    prod = _sc_pairprod(node_emb, i, j)

    x2, st_x2 = pl.pallas_call(
        _x2_body,
        grid=(NBLK,),
        in_specs=[
            pl.BlockSpec((BK, HN), lambda k: (k, 0)),
            pl.BlockSpec((HN, 32), lambda k: (0, 0)),
            pl.BlockSpec((1, 32), lambda k: (0, 0)),
        ],
        out_specs=[pl.BlockSpec((BK, 32), lambda k: (k, 0)),
                   pl.BlockSpec((2, 32), lambda k: (0, 0))],
        out_shape=[jax.ShapeDtypeStruct((E, 32), f32),
                   jax.ShapeDtypeStruct((2, 32), f32)],
        scratch_shapes=[pltpu.VMEM((2, 32), f32)],
    )(prod, w2t, b2r)

    st_x3 = pl.pallas_call(
        _stats_body,
        grid=(NBLK,),
        in_specs=[pl.BlockSpec((BK, 32), lambda k: (k, 0))],
        out_specs=pl.BlockSpec((2, 32), lambda k: (0, 0)),
        out_shape=jax.ShapeDtypeStruct((2, 32), f32),
        scratch_shapes=[pltpu.VMEM((2, 32), f32)],
    )(x3)

    y2, st_y2 = pl.pallas_call(
        _act2_body,
        grid=(NBLK,),
        in_specs=[
            pl.BlockSpec((BK, 32), lambda k: (k, 0)),
            pl.BlockSpec((2, 32), lambda k: (0, 0)),
            pl.BlockSpec((1, 32), lambda k: (0, 0)),
            pl.BlockSpec((1, 32), lambda k: (0, 0)),
        ],
        out_specs=[pl.BlockSpec((BK, HE), lambda k: (k, 0)),
                   pl.BlockSpec((2, HE), lambda k: (0, 0))],
        out_shape=[jax.ShapeDtypeStruct((E, HE), f32),
                   jax.ShapeDtypeStruct((2, HE), f32)],
        scratch_shapes=[pltpu.VMEM((2, HE), f32)],
    )(x2, st_x2, g_c2.reshape(1, 32), be_c2.reshape(1, 32))

    msg = pl.pallas_call(
        _msg_body,
        grid=(NBLK,),
        in_specs=[
            pl.BlockSpec((BK, 32), lambda k: (k, 0)),
            pl.BlockSpec((2, 32), lambda k: (0, 0)),
            pl.BlockSpec((1, 32), lambda k: (0, 0)),
            pl.BlockSpec((1, 32), lambda k: (0, 0)),
        ],
        out_specs=pl.BlockSpec((BK, HE), lambda k: (k, 0)),
        out_shape=jax.ShapeDtypeStruct((T, HE), jnp.bfloat16),
    )(x3, st_x3, g_c3.reshape(1, 32), be_c3.reshape(1, 32))

    zeros = jnp.zeros((ZR, HE), jnp.bfloat16)
    agg = _sc_scatter(msg, idx_ji, zeros)

    st_agg = pl.pallas_call(
        _stats_body,
        grid=(NBLK,),
        in_specs=[pl.BlockSpec((BK, HE), lambda k: (k, 0))],
        out_specs=pl.BlockSpec((2, HE), lambda k: (0, 0)),
        out_shape=jax.ShapeDtypeStruct((2, HE), f32),
        scratch_shapes=[pltpu.VMEM((2, HE), f32)],
    )(agg)

    out = pl.pallas_call(
        _final_body,
        grid=(NBLK,),
        in_specs=[
            pl.BlockSpec((BK, HE), lambda k: (k, 0)),
            pl.BlockSpec((BK, HE), lambda k: (k, 0)),
            pl.BlockSpec((2, HE), lambda k: (0, 0)),
            pl.BlockSpec((BK, HE), lambda k: (k, 0)),
            pl.BlockSpec((2, HE), lambda k: (0, 0)),
            pl.BlockSpec((1, HE), lambda k: (0, 0)),
            pl.BlockSpec((1, HE), lambda k: (0, 0)),
            pl.BlockSpec((1, HE), lambda k: (0, 0)),
            pl.BlockSpec((1, HE), lambda k: (0, 0)),
        ],
        out_specs=pl.BlockSpec((BK, HE), lambda k: (k, 0)),
        out_shape=jax.ShapeDtypeStruct((E, HE), f32),
    )(edge_emb, y2, st_y2, agg, st_agg,
      g_c2_2.reshape(1, HE), be_c2_2.reshape(1, HE),
      g_c3_2.reshape(1, HE), be_c3_2.reshape(1, HE))

    return out


# speedup vs baseline: 1.2224x; 1.0987x over previous
import functools

import jax
import jax.numpy as jnp
from jax import lax
from jax.experimental import pallas as pl
from jax.experimental.pallas import tpu as pltpu
from jax.experimental.pallas import tpu_sc as plsc

N = 10000
E = 320000
T = 320000
HN = 128
HE = 16
EPS = 1e-5

NC = 2
NS = 16
NW = NC * NS

BK = 8000
NBLK = E // BK

_mesh = plsc.VectorSubcoreMesh(core_axis_name="c", subcore_axis_name="s")
_sc_params = pltpu.CompilerParams(use_tc_tiling_on_sc=False)


def _nodeproj_body(x_ref, w_ref, pi_ref, pj_ref, pk_ref):
    r = jnp.dot(x_ref[...], w_ref[...], preferred_element_type=jnp.float32)
    pi_ref[...] = r[:, 0:32]
    pj_ref[...] = r[:, 32:64]
    pk_ref[...] = r[:, 64:96]


def _edgeproj_body(e_ref, wji_ref, wkj_ref, b_ref, qji_ref, qkj_ref):
    e = e_ref[...]
    qji_ref[...] = jnp.dot(e, wji_ref[...], preferred_element_type=jnp.float32) + b_ref[...]
    qkj_ref[...] = jnp.dot(e, wkj_ref[...], preferred_element_type=jnp.float32)


def _x2_body(prod_ref, w_ref, b_ref, x2_ref, st_ref, acc_ref):
    k = pl.program_id(0)
    x2 = jnp.dot(prod_ref[...], w_ref[...], preferred_element_type=jnp.float32) + b_ref[...]
    x2_ref[...] = x2

    @pl.when(k == 0)
    def _():
        acc_ref[...] = jnp.zeros_like(acc_ref)

    s = jnp.sum(x2, axis=0, keepdims=True)
    sq = jnp.sum(x2 * x2, axis=0, keepdims=True)
    acc_ref[...] = acc_ref[...] + jnp.concatenate([s, sq], axis=0)

    @pl.when(k == NBLK - 1)
    def _():
        st_ref[...] = acc_ref[...]


def _stats_body(x_ref, st_ref, acc_ref):
    k = pl.program_id(0)
    x = x_ref[...]

    @pl.when(k == 0)
    def _():
        acc_ref[...] = jnp.zeros_like(acc_ref)

    s = jnp.sum(x, axis=0, keepdims=True)
    sq = jnp.sum(x * x, axis=0, keepdims=True)
    acc_ref[...] = acc_ref[...] + jnp.concatenate([s, sq], axis=0)

    @pl.when(k == NBLK - 1)
    def _():
        st_ref[...] = acc_ref[...]


def _bn_act(x, st, g, b, nrows):
    mean = st[0:1, :] / nrows
    var = st[1:2, :] / nrows - mean * mean
    xn = g * (x - mean) * lax.rsqrt(var + EPS) + b
    f = xn[:, 0:HE]
    c = xn[:, HE:2 * HE]
    return jax.nn.sigmoid(f) * jnp.tanh(c)


def _act2_body(x2_ref, st_ref, g_ref, b_ref, y2_ref, sty_ref, acc_ref):
    k = pl.program_id(0)
    y = _bn_act(x2_ref[...], st_ref[...], g_ref[...], b_ref[...], float(E))
    y2_ref[...] = y

    @pl.when(k == 0)
    def _():
        acc_ref[...] = jnp.zeros_like(acc_ref)

    s = jnp.sum(y, axis=0, keepdims=True)
    sq = jnp.sum(y * y, axis=0, keepdims=True)
    acc_ref[...] = acc_ref[...] + jnp.concatenate([s, sq], axis=0)

    @pl.when(k == NBLK - 1)
    def _():
        sty_ref[...] = acc_ref[...]


def _msg_body(x3_ref, st_ref, g_ref, b_ref, msg_ref):
    msg_ref[...] = _bn_act(x3_ref[...], st_ref[...], g_ref[...], b_ref[...], float(T))


def _final_body(e_ref, y2_ref, sty_ref, agg_ref, sta_ref, g22_ref, b22_ref,
                g32_ref, b32_ref, out_ref):
    sty = sty_ref[...]
    m2 = sty[0:1, :] / E
    v2 = sty[1:2, :] / E - m2 * m2
    c2e = g22_ref[...] * (y2_ref[...] - m2) * lax.rsqrt(v2 + EPS) + b22_ref[...]
    sta = sta_ref[...]
    m3 = sta[0:1, :] / E
    v3 = sta[1:2, :] / E - m3 * m3
    c3e = g32_ref[...] * (agg_ref[...] - m3) * lax.rsqrt(v3 + EPS) + b32_ref[...]
    out_ref[...] = jnp.tanh(e_ref[...] + c2e + c3e)


KC = 200
RPW = T // NW
NCH_C = RPW // KC

KD = 40
NCH_D = RPW // KD


def _sc_gather5_body(pi, pj, pk, qji, qkj, ii, ij, ik, iji, ikj, out,
                     vi0, vi1, rr0, rr1, vout, si0, si1, sg0, sg1):
    w = lax.axis_index("s") * NC + lax.axis_index("c")
    tbls = (pi, pj, pk, qji, qkj)
    idxs = (ii, ij, ik, iji, ikj)

    def idx_descs(ch, vi, sem):
        base = w * RPW + ch * KC
        return [pltpu.make_async_copy(idxs[a].at[pl.ds(base, KC)],
                                      vi.at[a], sem)
                for a in range(5)]

    def gather_descs(vi, rr, sem):
        return [pltpu.make_async_copy(tbls[a].at[vi.at[a]], rr.at[a], sem)
                for a in range(5)]

    def fire(descs):
        for d in descs:
            d.start()

    def drain(descs):
        for d in descs:
            d.wait()

    def compute_store(ch, rr):
        def row(r, cr):
            for o in (0, 16):
                sl = pl.ds(o, 16)
                vout[r, sl] = (rr[0, r, sl] + rr[1, r, sl] + rr[2, r, sl]
                               + rr[3, r, sl] + rr[4, r, sl])
            return cr
        lax.fori_loop(0, KC, row, 0)
        base = w * RPW + ch * KC
        pltpu.sync_copy(vout, out.at[pl.ds(base, KC)])

    fire(idx_descs(0, vi0, si0))
    drain(idx_descs(0, vi0, si0))
    fire(gather_descs(vi0, rr0, sg0))
    fire(idx_descs(1, vi1, si1))

    def pair(h, carry):
        ch_a = 2 * h
        drain(gather_descs(vi0, rr0, sg0))
        nxt_a = jnp.minimum(ch_a + 2, NCH_C - 2)
        fire(idx_descs(nxt_a, vi0, si0))
        drain(idx_descs(ch_a + 1, vi1, si1))
        fire(gather_descs(vi1, rr1, sg1))
        compute_store(ch_a, rr0)
        drain(idx_descs(nxt_a, vi0, si0))
        fire(gather_descs(vi0, rr0, sg0))
        drain(gather_descs(vi1, rr1, sg1))
        nxt_b = jnp.minimum(ch_a + 3, NCH_C - 1)
        fire(idx_descs(nxt_b, vi1, si1))
        compute_store(ch_a + 1, rr1)
        return carry

    lax.fori_loop(0, NCH_C // 2, pair, 0)
    drain(gather_descs(vi0, rr0, sg0))
    drain(idx_descs(NCH_C - 1, vi1, si1))


def _sc_pairprod_body(tbl, ii, ij, out, vi0, vi1, rr0, rr1, vout,
                      si0, si1, sg0, sg1):
    w = lax.axis_index("s") * NC + lax.axis_index("c")
    idxs = (ii, ij)

    def idx_descs(ch, vi, sem):
        base = w * RPW + ch * KD
        return [pltpu.make_async_copy(idxs[a].at[pl.ds(base, KD)],
                                      vi.at[a], sem)
                for a in range(2)]

    def gather_descs(vi, rr, sem):
        return [pltpu.make_async_copy(tbl.at[vi.at[a]], rr.at[a], sem)
                for a in range(2)]

    def fire(descs):
        for d in descs:
            d.start()

    def drain(descs):
        for d in descs:
            d.wait()

    def compute_store(ch, rr):
        def row(r, cr):
            for o in range(0, HN, 16):
                sl = pl.ds(o, 16)
                vout[r, sl] = rr[0, r, sl] * rr[1, r, sl]
            return cr
        lax.fori_loop(0, KD, row, 0)
        base = w * RPW + ch * KD
        pltpu.sync_copy(vout, out.at[pl.ds(base, KD)])

    fire(idx_descs(0, vi0, si0))
    drain(idx_descs(0, vi0, si0))
    fire(gather_descs(vi0, rr0, sg0))
    fire(idx_descs(1, vi1, si1))

    def pair(h, carry):
        ch_a = 2 * h
        drain(gather_descs(vi0, rr0, sg0))
        nxt_a = jnp.minimum(ch_a + 2, NCH_D - 2)
        fire(idx_descs(nxt_a, vi0, si0))
        drain(idx_descs(ch_a + 1, vi1, si1))
        fire(gather_descs(vi1, rr1, sg1))
        compute_store(ch_a, rr0)
        drain(idx_descs(nxt_a, vi0, si0))
        fire(gather_descs(vi0, rr0, sg0))
        drain(gather_descs(vi1, rr1, sg1))
        nxt_b = jnp.minimum(ch_a + 3, NCH_D - 1)
        fire(idx_descs(nxt_b, vi1, si1))
        compute_store(ch_a + 1, rr1)
        return carry

    lax.fori_loop(0, NCH_D // 2, pair, 0)
    drain(gather_descs(vi0, rr0, sg0))
    drain(idx_descs(NCH_D - 1, vi1, si1))


CH = 80000
DR = 128
SHR = CH + DR
ZR = SHR // NS
TPS = T // NS
BK3 = 2000
NSUB = BK3 // 80


def _sc_scatter_body(msg, iji, zeros, out, shr, vidx, vclamp, vmsg, sem):
    c = lax.axis_index("c")
    s = lax.axis_index("s")
    for r in range(2):
        chunk_id = r * NC + c
        lo = chunk_id * CH
        pltpu.sync_copy(zeros, shr.at[pl.ds(s * ZR, ZR)])
        plsc.subcore_barrier()

        def big(b, carry):
            bb = s * TPS + b * BK3
            d0 = pltpu.async_copy(iji.at[pl.ds(bb, BK3)], vidx, sem)
            d1 = pltpu.async_copy(msg.at[pl.ds(bb, BK3)], vmsg, sem)
            d0.wait(); d1.wait()

            def sub(j, cr):
                for o in range(5):
                    v = vidx[pl.ds(j * 80 + o * 16, 16)]
                    inr = (v >= lo) & (v < lo + CH)
                    dmp = CH + (v & (DR - 1))
                    vclamp[j, pl.ds(o * 16, 16)] = jnp.where(inr, v - lo, dmp)
                return cr

            lax.fori_loop(0, NSUB, sub, 0)

            def scat_descs(j):
                return pltpu.make_async_copy(vmsg.at[pl.ds(j * 80, 80)],
                                             shr.at[vclamp.at[j]], sem)

            def scat_fire(j, cr):
                scat_descs(j).start(add=True)
                return cr

            def scat_drain(j, cr):
                scat_descs(j).wait()
                return cr

            lax.fori_loop(0, NSUB, scat_fire, 0)
            lax.fori_loop(0, NSUB, scat_drain, 0)
            return carry

        lax.fori_loop(0, TPS // BK3, big, 0)
        plsc.subcore_barrier()
        cpr = CH // NS
        pltpu.sync_copy(shr.at[pl.ds(s * cpr, cpr)],
                        out.at[pl.ds(lo + s * cpr, cpr)])
        plsc.subcore_barrier()


_sc_gather5 = functools.partial(
    pl.kernel,
    out_type=jax.ShapeDtypeStruct((T, 32), jnp.float32),
    mesh=_mesh,
    compiler_params=_sc_params,
    scratch_types=(
        [pltpu.VMEM((5, KC), jnp.int32)] * 2
        + [pltpu.VMEM((5, KC, 32), jnp.float32)] * 2
        + [pltpu.VMEM((KC, 32), jnp.float32)]
        + [pltpu.SemaphoreType.DMA] * 4
    ),
)(_sc_gather5_body)

_sc_pairprod = functools.partial(
    pl.kernel,
    out_type=jax.ShapeDtypeStruct((E, HN), jnp.float32),
    mesh=_mesh,
    compiler_params=_sc_params,
    scratch_types=(
        [pltpu.VMEM((2, KD), jnp.int32)] * 2
        + [pltpu.VMEM((2, KD, HN), jnp.float32)] * 2
        + [pltpu.VMEM((KD, HN), jnp.float32)]
        + [pltpu.SemaphoreType.DMA] * 4
    ),
)(_sc_pairprod_body)

_sc_scatter = functools.partial(
    pl.kernel,
    out_type=jax.ShapeDtypeStruct((E, HE), jnp.float32),
    mesh=_mesh,
    compiler_params=_sc_params,
    scratch_types=(
        [pltpu.VMEM_SHARED((SHR, HE), jnp.float32),
         pltpu.VMEM((BK3,), jnp.int32),
         pltpu.VMEM((NSUB, 80), jnp.int32),
         pltpu.VMEM((BK3, HE), jnp.float32),
         pltpu.SemaphoreType.DMA]
    ),
)(_sc_scatter_body)


def kernel(node_emb, edge_emb, i, j, idx_i, idx_j, idx_k, idx_ji, idx_kj,
           W_c2, b_c2, W_c3, b_c3,
           g_c2, be_c2, g_c3, be_c3,
           g_c2_2, be_c2_2, g_c3_2, be_c3_2):
    f32 = jnp.float32
    wn = jnp.concatenate([W_c3[:, 0:HN].T, W_c3[:, HN:2 * HN].T,
                          W_c3[:, 2 * HN:3 * HN].T], axis=1)
    wji = W_c3[:, 3 * HN:3 * HN + HE].T
    wkj = W_c3[:, 3 * HN + HE:3 * HN + 2 * HE].T
    b3r = b_c3.reshape(1, 32)
    w2t = W_c2.T
    b2r = b_c2.reshape(1, 32)

    p_i, p_j, p_k = pl.pallas_call(
        _nodeproj_body,
        out_shape=[jax.ShapeDtypeStruct((N, 32), f32)] * 3,
    )(node_emb, wn)

    q_ji, q_kj = pl.pallas_call(
        _edgeproj_body,
        grid=(NBLK,),
        in_specs=[
            pl.BlockSpec((BK, HE), lambda k: (k, 0)),
            pl.BlockSpec((HE, 32), lambda k: (0, 0)),
            pl.BlockSpec((HE, 32), lambda k: (0, 0)),
            pl.BlockSpec((1, 32), lambda k: (0, 0)),
        ],
        out_specs=[pl.BlockSpec((BK, 32), lambda k: (k, 0))] * 2,
        out_shape=[jax.ShapeDtypeStruct((E, 32), f32)] * 2,
    )(edge_emb, wji, wkj, b3r)

    x3 = _sc_gather5(p_i, p_j, p_k, q_ji, q_kj, idx_i, idx_j, idx_k,
                     idx_ji, idx_kj)

    prod = _sc_pairprod(node_emb, i, j)

    x2, st_x2 = pl.pallas_call(
        _x2_body,
        grid=(NBLK,),
        in_specs=[
            pl.BlockSpec((BK, HN), lambda k: (k, 0)),
            pl.BlockSpec((HN, 32), lambda k: (0, 0)),
            pl.BlockSpec((1, 32), lambda k: (0, 0)),
        ],
        out_specs=[pl.BlockSpec((BK, 32), lambda k: (k, 0)),
                   pl.BlockSpec((2, 32), lambda k: (0, 0))],
        out_shape=[jax.ShapeDtypeStruct((E, 32), f32),
                   jax.ShapeDtypeStruct((2, 32), f32)],
        scratch_shapes=[pltpu.VMEM((2, 32), f32)],
    )(prod, w2t, b2r)

    st_x3 = pl.pallas_call(
        _stats_body,
        grid=(NBLK,),
        in_specs=[pl.BlockSpec((BK, 32), lambda k: (k, 0))],
        out_specs=pl.BlockSpec((2, 32), lambda k: (0, 0)),
        out_shape=jax.ShapeDtypeStruct((2, 32), f32),
        scratch_shapes=[pltpu.VMEM((2, 32), f32)],
    )(x3)

    y2, st_y2 = pl.pallas_call(
        _act2_body,
        grid=(NBLK,),
        in_specs=[
            pl.BlockSpec((BK, 32), lambda k: (k, 0)),
            pl.BlockSpec((2, 32), lambda k: (0, 0)),
            pl.BlockSpec((1, 32), lambda k: (0, 0)),
            pl.BlockSpec((1, 32), lambda k: (0, 0)),
        ],
        out_specs=[pl.BlockSpec((BK, HE), lambda k: (k, 0)),
                   pl.BlockSpec((2, HE), lambda k: (0, 0))],
        out_shape=[jax.ShapeDtypeStruct((E, HE), f32),
                   jax.ShapeDtypeStruct((2, HE), f32)],
        scratch_shapes=[pltpu.VMEM((2, HE), f32)],
    )(x2, st_x2, g_c2.reshape(1, 32), be_c2.reshape(1, 32))

    msg = pl.pallas_call(
        _msg_body,
        grid=(NBLK,),
        in_specs=[
            pl.BlockSpec((BK, 32), lambda k: (k, 0)),
            pl.BlockSpec((2, 32), lambda k: (0, 0)),
            pl.BlockSpec((1, 32), lambda k: (0, 0)),
            pl.BlockSpec((1, 32), lambda k: (0, 0)),
        ],
        out_specs=pl.BlockSpec((BK, HE), lambda k: (k, 0)),
        out_shape=jax.ShapeDtypeStruct((T, HE), f32),
    )(x3, st_x3, g_c3.reshape(1, 32), be_c3.reshape(1, 32))

    zeros = jnp.zeros((ZR, HE), f32)
    agg = _sc_scatter(msg, idx_ji, zeros)

    st_agg = pl.pallas_call(
        _stats_body,
        grid=(NBLK,),
        in_specs=[pl.BlockSpec((BK, HE), lambda k: (k, 0))],
        out_specs=pl.BlockSpec((2, HE), lambda k: (0, 0)),
        out_shape=jax.ShapeDtypeStruct((2, HE), f32),
        scratch_shapes=[pltpu.VMEM((2, HE), f32)],
    )(agg)

    out = pl.pallas_call(
        _final_body,
        grid=(NBLK,),
        in_specs=[
            pl.BlockSpec((BK, HE), lambda k: (k, 0)),
            pl.BlockSpec((BK, HE), lambda k: (k, 0)),
            pl.BlockSpec((2, HE), lambda k: (0, 0)),
            pl.BlockSpec((BK, HE), lambda k: (k, 0)),
            pl.BlockSpec((2, HE), lambda k: (0, 0)),
            pl.BlockSpec((1, HE), lambda k: (0, 0)),
            pl.BlockSpec((1, HE), lambda k: (0, 0)),
            pl.BlockSpec((1, HE), lambda k: (0, 0)),
            pl.BlockSpec((1, HE), lambda k: (0, 0)),
        ],
        out_specs=pl.BlockSpec((BK, HE), lambda k: (k, 0)),
        out_shape=jax.ShapeDtypeStruct((E, HE), f32),
    )(edge_emb, y2, st_y2, agg, st_agg,
      g_c2_2.reshape(1, HE), be_c2_2.reshape(1, HE),
      g_c3_2.reshape(1, HE), be_c3_2.reshape(1, HE))

    return out


# speedup vs baseline: 1.2232x; 1.0007x over previous
import functools

import jax
import jax.numpy as jnp
from jax import lax
from jax.experimental import pallas as pl
from jax.experimental.pallas import tpu as pltpu
from jax.experimental.pallas import tpu_sc as plsc

N = 10000
E = 320000
T = 320000
HN = 128
HE = 16
EPS = 1e-5

NC = 2
NS = 16
NW = NC * NS

BK = 8000
NBLK = E // BK

_mesh = plsc.VectorSubcoreMesh(core_axis_name="c", subcore_axis_name="s")
_sc_params = pltpu.CompilerParams(use_tc_tiling_on_sc=False)


def _nodeproj_body(x_ref, w_ref, pi_ref, pj_ref, pk_ref):
    r = jnp.dot(x_ref[...], w_ref[...], preferred_element_type=jnp.float32)
    pi_ref[...] = r[:, 0:32]
    pj_ref[...] = r[:, 32:64]
    pk_ref[...] = r[:, 64:96]


def _edgeproj_body(e_ref, wji_ref, wkj_ref, b_ref, qji_ref, qkj_ref):
    e = e_ref[...]
    qji_ref[...] = jnp.dot(e, wji_ref[...], preferred_element_type=jnp.float32) + b_ref[...]
    qkj_ref[...] = jnp.dot(e, wkj_ref[...], preferred_element_type=jnp.float32)


def _x2_body(prod_ref, w_ref, b_ref, x2_ref, st_ref, acc_ref):
    k = pl.program_id(0)
    x2 = jnp.dot(prod_ref[...], w_ref[...], preferred_element_type=jnp.float32) + b_ref[...]
    x2_ref[...] = x2

    @pl.when(k == 0)
    def _():
        acc_ref[...] = jnp.zeros_like(acc_ref)

    s = jnp.sum(x2, axis=0, keepdims=True)
    sq = jnp.sum(x2 * x2, axis=0, keepdims=True)
    acc_ref[...] = acc_ref[...] + jnp.concatenate([s, sq], axis=0)

    @pl.when(k == NBLK - 1)
    def _():
        st_ref[...] = acc_ref[...]


def _stats_body(x_ref, st_ref, acc_ref):
    k = pl.program_id(0)
    x = x_ref[...]

    @pl.when(k == 0)
    def _():
        acc_ref[...] = jnp.zeros_like(acc_ref)

    s = jnp.sum(x, axis=0, keepdims=True)
    sq = jnp.sum(x * x, axis=0, keepdims=True)
    acc_ref[...] = acc_ref[...] + jnp.concatenate([s, sq], axis=0)

    @pl.when(k == NBLK - 1)
    def _():
        st_ref[...] = acc_ref[...]


def _bn_act(x, st, g, b, nrows):
    mean = st[0:1, :] / nrows
    var = st[1:2, :] / nrows - mean * mean
    xn = g * (x - mean) * lax.rsqrt(var + EPS) + b
    f = xn[:, 0:HE]
    c = xn[:, HE:2 * HE]
    return jax.nn.sigmoid(f) * jnp.tanh(c)


def _act2_body(x2_ref, st_ref, g_ref, b_ref, y2_ref, sty_ref, acc_ref):
    k = pl.program_id(0)
    y = _bn_act(x2_ref[...], st_ref[...], g_ref[...], b_ref[...], float(E))
    y2_ref[...] = y

    @pl.when(k == 0)
    def _():
        acc_ref[...] = jnp.zeros_like(acc_ref)

    s = jnp.sum(y, axis=0, keepdims=True)
    sq = jnp.sum(y * y, axis=0, keepdims=True)
    acc_ref[...] = acc_ref[...] + jnp.concatenate([s, sq], axis=0)

    @pl.when(k == NBLK - 1)
    def _():
        sty_ref[...] = acc_ref[...]


def _msg_body(x3_ref, st_ref, g_ref, b_ref, msg_ref):
    msg_ref[...] = _bn_act(x3_ref[...], st_ref[...], g_ref[...], b_ref[...], float(T))


def _final_body(e_ref, y2_ref, sty_ref, agg_ref, sta_ref, g22_ref, b22_ref,
                g32_ref, b32_ref, out_ref):
    sty = sty_ref[...]
    m2 = sty[0:1, :] / E
    v2 = sty[1:2, :] / E - m2 * m2
    c2e = g22_ref[...] * (y2_ref[...] - m2) * lax.rsqrt(v2 + EPS) + b22_ref[...]
    sta = sta_ref[...]
    m3 = sta[0:1, :] / E
    v3 = sta[1:2, :] / E - m3 * m3
    c3e = g32_ref[...] * (agg_ref[...] - m3) * lax.rsqrt(v3 + EPS) + b32_ref[...]
    out_ref[...] = jnp.tanh(e_ref[...] + c2e + c3e)


KC = 200
RPW = T // NW
NCH_C = RPW // KC

KD = 40
NCH_D = RPW // KD


def _sc_gather5_body(pi, pj, pk, qji, qkj, ii, ij, ik, iji, ikj, out,
                     vi0, vi1, rr0, rr1, vout, si0, si1, sg0, sg1):
    w = lax.axis_index("s") * NC + lax.axis_index("c")
    tbls = (pi, pj, pk, qji, qkj)
    idxs = (ii, ij, ik, iji, ikj)

    def idx_descs(ch, vi, sem):
        base = w * RPW + ch * KC
        return [pltpu.make_async_copy(idxs[a].at[pl.ds(base, KC)],
                                      vi.at[a], sem)
                for a in range(5)]

    def gather_descs(vi, rr, sem):
        return [pltpu.make_async_copy(tbls[a].at[vi.at[a]], rr.at[a], sem)
                for a in range(5)]

    def fire(descs):
        for d in descs:
            d.start()

    def drain(descs):
        for d in descs:
            d.wait()

    def compute_store(ch, rr):
        def row(r, cr):
            for o in (0, 16):
                sl = pl.ds(o, 16)
                vout[r, sl] = (rr[0, r, sl] + rr[1, r, sl] + rr[2, r, sl]
                               + rr[3, r, sl] + rr[4, r, sl])
            return cr
        lax.fori_loop(0, KC, row, 0)
        base = w * RPW + ch * KC
        pltpu.sync_copy(vout, out.at[pl.ds(base, KC)])

    fire(idx_descs(0, vi0, si0))
    drain(idx_descs(0, vi0, si0))
    fire(gather_descs(vi0, rr0, sg0))
    fire(idx_descs(1, vi1, si1))

    def pair(h, carry):
        ch_a = 2 * h
        drain(gather_descs(vi0, rr0, sg0))
        nxt_a = jnp.minimum(ch_a + 2, NCH_C - 2)
        fire(idx_descs(nxt_a, vi0, si0))
        drain(idx_descs(ch_a + 1, vi1, si1))
        fire(gather_descs(vi1, rr1, sg1))
        compute_store(ch_a, rr0)
        drain(idx_descs(nxt_a, vi0, si0))
        fire(gather_descs(vi0, rr0, sg0))
        drain(gather_descs(vi1, rr1, sg1))
        nxt_b = jnp.minimum(ch_a + 3, NCH_C - 1)
        fire(idx_descs(nxt_b, vi1, si1))
        compute_store(ch_a + 1, rr1)
        return carry

    lax.fori_loop(0, NCH_C // 2, pair, 0)
    drain(gather_descs(vi0, rr0, sg0))
    drain(idx_descs(NCH_C - 1, vi1, si1))


def _sc_pairprod_body(tbl, ii, ij, out, vi0, vi1, rr0, rr1, vout,
                      si0, si1, sg0, sg1):
    w = lax.axis_index("s") * NC + lax.axis_index("c")
    idxs = (ii, ij)

    def idx_descs(ch, vi, sem):
        base = w * RPW + ch * KD
        return [pltpu.make_async_copy(idxs[a].at[pl.ds(base, KD)],
                                      vi.at[a], sem)
                for a in range(2)]

    def gather_descs(vi, rr, sem):
        return [pltpu.make_async_copy(tbl.at[vi.at[a]], rr.at[a], sem)
                for a in range(2)]

    def fire(descs):
        for d in descs:
            d.start()

    def drain(descs):
        for d in descs:
            d.wait()

    def compute_store(ch, rr):
        def row(r, cr):
            for o in range(0, HN, 16):
                sl = pl.ds(o, 16)
                vout[r, sl] = rr[0, r, sl] * rr[1, r, sl]
            return cr
        lax.fori_loop(0, KD, row, 0)
        base = w * RPW + ch * KD
        pltpu.sync_copy(vout, out.at[pl.ds(base, KD)])

    fire(idx_descs(0, vi0, si0))
    drain(idx_descs(0, vi0, si0))
    fire(gather_descs(vi0, rr0, sg0))
    fire(idx_descs(1, vi1, si1))

    def pair(h, carry):
        ch_a = 2 * h
        drain(gather_descs(vi0, rr0, sg0))
        nxt_a = jnp.minimum(ch_a + 2, NCH_D - 2)
        fire(idx_descs(nxt_a, vi0, si0))
        drain(idx_descs(ch_a + 1, vi1, si1))
        fire(gather_descs(vi1, rr1, sg1))
        compute_store(ch_a, rr0)
        drain(idx_descs(nxt_a, vi0, si0))
        fire(gather_descs(vi0, rr0, sg0))
        drain(gather_descs(vi1, rr1, sg1))
        nxt_b = jnp.minimum(ch_a + 3, NCH_D - 1)
        fire(idx_descs(nxt_b, vi1, si1))
        compute_store(ch_a + 1, rr1)
        return carry

    lax.fori_loop(0, NCH_D // 2, pair, 0)
    drain(gather_descs(vi0, rr0, sg0))
    drain(idx_descs(NCH_D - 1, vi1, si1))


CH = 80000
DR = 128
SHR = CH + DR
ZR = SHR // NS
TPS = T // NS
BK3 = 2000
NSUB = BK3 // 80


def _sc_scatter_body(msg, iji, zeros, out, shr, vidx, vclamp, vmsg, sem):
    c = lax.axis_index("c")
    s = lax.axis_index("s")
    for r in range(2):
        chunk_id = r * NC + c
        lo = chunk_id * CH
        pltpu.sync_copy(zeros, shr.at[pl.ds(s * ZR, ZR)])
        plsc.subcore_barrier()

        def big(b, carry):
            bb = s * TPS + b * BK3
            d0 = pltpu.async_copy(iji.at[pl.ds(bb, BK3)], vidx, sem)
            d1 = pltpu.async_copy(msg.at[pl.ds(bb, BK3)], vmsg, sem)
            d0.wait(); d1.wait()

            def sub(j, cr):
                for o in range(5):
                    v = vidx[pl.ds(j * 80 + o * 16, 16)]
                    inr = (v >= lo) & (v < lo + CH)
                    dmp = CH + (v & (DR - 1))
                    vclamp[j, pl.ds(o * 16, 16)] = jnp.where(inr, v - lo, dmp)
                return cr

            lax.fori_loop(0, NSUB, sub, 0)

            def scat_descs(j):
                return pltpu.make_async_copy(vmsg.at[pl.ds(j * 80, 80)],
                                             shr.at[vclamp.at[j]], sem)

            def scat_fire(j, cr):
                scat_descs(j).start(add=True)
                return cr

            def scat_drain(j, cr):
                scat_descs(j).wait()
                return cr

            lax.fori_loop(0, NSUB, scat_fire, 0)
            lax.fori_loop(0, NSUB, scat_drain, 0)
            return carry

        lax.fori_loop(0, TPS // BK3, big, 0)
        plsc.subcore_barrier()
        cpr = CH // NS
        pltpu.sync_copy(shr.at[pl.ds(s * cpr, cpr)],
                        out.at[pl.ds(lo + s * cpr, cpr)])
        plsc.subcore_barrier()


_sc_gather5 = functools.partial(
    pl.kernel,
    out_type=jax.ShapeDtypeStruct((T, 32), jnp.float32),
    mesh=_mesh,
    compiler_params=_sc_params,
    scratch_types=(
        [pltpu.VMEM((5, KC), jnp.int32)] * 2
        + [pltpu.VMEM((5, KC, 32), jnp.float32)] * 2
        + [pltpu.VMEM((KC, 32), jnp.float32)]
        + [pltpu.SemaphoreType.DMA] * 4
    ),
)(_sc_gather5_body)

_sc_pairprod = functools.partial(
    pl.kernel,
    out_type=jax.ShapeDtypeStruct((E, HN), jnp.float32),
    mesh=_mesh,
    compiler_params=_sc_params,
    scratch_types=(
        [pltpu.VMEM((2, KD), jnp.int32)] * 2
        + [pltpu.VMEM((2, KD, HN), jnp.float32)] * 2
        + [pltpu.VMEM((KD, HN), jnp.float32)]
        + [pltpu.SemaphoreType.DMA] * 4
    ),
)(_sc_pairprod_body)

_sc_scatter = functools.partial(
    pl.kernel,
    out_type=jax.ShapeDtypeStruct((E, HE), jnp.float32),
    mesh=_mesh,
    compiler_params=_sc_params,
    scratch_types=(
        [pltpu.VMEM_SHARED((SHR, HE), jnp.float32),
         pltpu.VMEM((BK3,), jnp.int32),
         pltpu.VMEM((NSUB, 80), jnp.int32),
         pltpu.VMEM((BK3, HE), jnp.float32),
         pltpu.SemaphoreType.DMA]
    ),
)(_sc_scatter_body)


def kernel(node_emb, edge_emb, i, j, idx_i, idx_j, idx_k, idx_ji, idx_kj,
           W_c2, b_c2, W_c3, b_c3,
           g_c2, be_c2, g_c3, be_c3,
           g_c2_2, be_c2_2, g_c3_2, be_c3_2):
    f32 = jnp.float32
    wn = jnp.concatenate([W_c3[:, 0:HN].T, W_c3[:, HN:2 * HN].T,
                          W_c3[:, 2 * HN:3 * HN].T], axis=1)
    wji = W_c3[:, 3 * HN:3 * HN + HE].T
    wkj = W_c3[:, 3 * HN + HE:3 * HN + 2 * HE].T
    b3r = b_c3.reshape(1, 32)
    w2t = W_c2.T
    b2r = b_c2.reshape(1, 32)

    p_i, p_j, p_k = pl.pallas_call(
        _nodeproj_body,
        out_shape=[jax.ShapeDtypeStruct((N, 32), f32)] * 3,
    )(node_emb, wn)

    q_ji, q_kj = pl.pallas_call(
        _edgeproj_body,
        grid=(NBLK,),
        in_specs=[
            pl.BlockSpec((BK, HE), lambda k: (k, 0)),
            pl.BlockSpec((HE, 32), lambda k: (0, 0)),
            pl.BlockSpec((HE, 32), lambda k: (0, 0)),
            pl.BlockSpec((1, 32), lambda k: (0, 0)),
        ],
        out_specs=[pl.BlockSpec((BK, 32), lambda k: (k, 0))] * 2,
        out_shape=[jax.ShapeDtypeStruct((E, 32), f32)] * 2,
    )(edge_emb, wji, wkj, b3r)

    x3 = _sc_gather5(p_i, p_j, p_k, q_ji, q_kj, idx_i, idx_j, idx_k,
                     idx_ji, idx_kj)

    prod = _sc_pairprod(node_emb, i, j)

    st_x3 = pl.pallas_call(
        _stats_body,
        grid=(NBLK,),
        in_specs=[pl.BlockSpec((BK, 32), lambda k: (k, 0))],
        out_specs=pl.BlockSpec((2, 32), lambda k: (0, 0)),
        out_shape=jax.ShapeDtypeStruct((2, 32), f32),
        scratch_shapes=[pltpu.VMEM((2, 32), f32)],
    )(x3)

    msg = pl.pallas_call(
        _msg_body,
        grid=(NBLK,),
        in_specs=[
            pl.BlockSpec((BK, 32), lambda k: (k, 0)),
            pl.BlockSpec((2, 32), lambda k: (0, 0)),
            pl.BlockSpec((1, 32), lambda k: (0, 0)),
            pl.BlockSpec((1, 32), lambda k: (0, 0)),
        ],
        out_specs=pl.BlockSpec((BK, HE), lambda k: (k, 0)),
        out_shape=jax.ShapeDtypeStruct((T, HE), f32),
    )(x3, st_x3, g_c3.reshape(1, 32), be_c3.reshape(1, 32))

    zeros = jnp.zeros((ZR, HE), f32)
    agg = _sc_scatter(msg, idx_ji, zeros)

    x2, st_x2 = pl.pallas_call(
        _x2_body,
        grid=(NBLK,),
        in_specs=[
            pl.BlockSpec((BK, HN), lambda k: (k, 0)),
            pl.BlockSpec((HN, 32), lambda k: (0, 0)),
            pl.BlockSpec((1, 32), lambda k: (0, 0)),
        ],
        out_specs=[pl.BlockSpec((BK, 32), lambda k: (k, 0)),
                   pl.BlockSpec((2, 32), lambda k: (0, 0))],
        out_shape=[jax.ShapeDtypeStruct((E, 32), f32),
                   jax.ShapeDtypeStruct((2, 32), f32)],
        scratch_shapes=[pltpu.VMEM((2, 32), f32)],
    )(prod, w2t, b2r)

    y2, st_y2 = pl.pallas_call(
        _act2_body,
        grid=(NBLK,),
        in_specs=[
            pl.BlockSpec((BK, 32), lambda k: (k, 0)),
            pl.BlockSpec((2, 32), lambda k: (0, 0)),
            pl.BlockSpec((1, 32), lambda k: (0, 0)),
            pl.BlockSpec((1, 32), lambda k: (0, 0)),
        ],
        out_specs=[pl.BlockSpec((BK, HE), lambda k: (k, 0)),
                   pl.BlockSpec((2, HE), lambda k: (0, 0))],
        out_shape=[jax.ShapeDtypeStruct((E, HE), f32),
                   jax.ShapeDtypeStruct((2, HE), f32)],
        scratch_shapes=[pltpu.VMEM((2, HE), f32)],
    )(x2, st_x2, g_c2.reshape(1, 32), be_c2.reshape(1, 32))

    st_agg = pl.pallas_call(
        _stats_body,
        grid=(NBLK,),
        in_specs=[pl.BlockSpec((BK, HE), lambda k: (k, 0))],
        out_specs=pl.BlockSpec((2, HE), lambda k: (0, 0)),
        out_shape=jax.ShapeDtypeStruct((2, HE), f32),
        scratch_shapes=[pltpu.VMEM((2, HE), f32)],
    )(agg)

    out = pl.pallas_call(
        _final_body,
        grid=(NBLK,),
        in_specs=[
            pl.BlockSpec((BK, HE), lambda k: (k, 0)),
            pl.BlockSpec((BK, HE), lambda k: (k, 0)),
            pl.BlockSpec((2, HE), lambda k: (0, 0)),
            pl.BlockSpec((BK, HE), lambda k: (k, 0)),
            pl.BlockSpec((2, HE), lambda k: (0, 0)),
            pl.BlockSpec((1, HE), lambda k: (0, 0)),
            pl.BlockSpec((1, HE), lambda k: (0, 0)),
            pl.BlockSpec((1, HE), lambda k: (0, 0)),
            pl.BlockSpec((1, HE), lambda k: (0, 0)),
        ],
        out_specs=pl.BlockSpec((BK, HE), lambda k: (k, 0)),
        out_shape=jax.ShapeDtypeStruct((E, HE), f32),
    )(edge_emb, y2, st_y2, agg, st_agg,
      g_c2_2.reshape(1, HE), be_c2_2.reshape(1, HE),
      g_c3_2.reshape(1, HE), be_c3_2.reshape(1, HE))

    return out
